# baseline (speedup 1.0000x reference)
import functools

import jax
import jax.numpy as jnp
from jax import lax
from jax.experimental import pallas as pl
from jax.experimental.pallas import tpu as pltpu
from jax.experimental.pallas import tpu_sc as plsc

NROWS = 128
NCOLS = 100000
NUPD = 16384
NC = 2
NS = 16
CHUNK = 128
NCHUNK = NUPD // CHUNK
CPT = NCHUNK // NS
ROWS_PER_CORE = NROWS // NC

COPY = 6256
COPY_LAST = NCOLS - (NS - 1) * COPY


def _body(x_hbm, idx_hbm, src_hbm, out_hbm, buf, idx_v, src_v, stage_v):
  c = lax.axis_index("c")
  s = lax.axis_index("s")
  off = s * COPY

  pltpu.sync_copy(idx_hbm.at[pl.ds(s * CPT, CPT)], idx_v)

  def row_step(i, carry):
    r = c * ROWS_PER_CORE + i
    base = r * NCOLS

    @pl.when(s < NS - 1)
    def _():
      pltpu.sync_copy(x_hbm.at[pl.ds(base + off, COPY)], stage_v)
      pltpu.sync_copy(stage_v, buf.at[pl.ds(off, COPY)])

    @pl.when(s == NS - 1)
    def _():
      pltpu.sync_copy(x_hbm.at[pl.ds(base + off, COPY_LAST)],
                      stage_v.at[pl.ds(0, COPY_LAST)])
      pltpu.sync_copy(stage_v.at[pl.ds(0, COPY_LAST)],
                      buf.at[pl.ds(off, COPY_LAST)])

    pltpu.sync_copy(src_hbm.at[r, pl.ds(s * CPT, CPT)], src_v)

    plsc.subcore_barrier()

    for j in range(CPT):
      pltpu.sync_copy(src_v.at[j], buf.at[idx_v.at[j]], add=True)

    plsc.subcore_barrier()

    @pl.when(s < NS - 1)
    def _():
      pltpu.sync_copy(buf.at[pl.ds(off, COPY)], stage_v)
      pltpu.sync_copy(stage_v, out_hbm.at[pl.ds(base + off, COPY)])

    @pl.when(s == NS - 1)
    def _():
      pltpu.sync_copy(buf.at[pl.ds(off, COPY_LAST)],
                      stage_v.at[pl.ds(0, COPY_LAST)])
      pltpu.sync_copy(stage_v.at[pl.ds(0, COPY_LAST)],
                      out_hbm.at[pl.ds(base + off, COPY_LAST)])

    plsc.subcore_barrier()
    return carry

  lax.fori_loop(0, ROWS_PER_CORE, row_step, 0)


@jax.jit
def kernel(x, indices, src):
  idx = indices.astype(jnp.int32).reshape(NCHUNK, CHUNK)
  src3 = src.reshape(NROWS, NCHUNK, CHUNK)
  x_flat = x.reshape(NROWS * NCOLS)

  mesh = plsc.VectorSubcoreMesh(
      core_axis_name="c", subcore_axis_name="s", num_cores=NC, num_subcores=NS)
  run = pl.kernel(
      _body,
      out_type=jax.ShapeDtypeStruct((NROWS * NCOLS,), jnp.float32),
      mesh=mesh,
      scratch_types=[
          pltpu.VMEM_SHARED((NCOLS,), jnp.float32),
          pltpu.VMEM((CPT, CHUNK), jnp.int32),
          pltpu.VMEM((CPT, CHUNK), jnp.float32),
          pltpu.VMEM((COPY,), jnp.float32),
      ],
  )
  out = run(x_flat, idx, src3)
  return out.reshape(NROWS, NCOLS)


# speedup vs baseline: 1.2679x; 1.2679x over previous
import functools

import jax
import jax.numpy as jnp
from jax import lax
from jax.experimental import pallas as pl
from jax.experimental.pallas import tpu as pltpu
from jax.experimental.pallas import tpu_sc as plsc

NROWS = 128
NCOLS = 100000
NUPD = 16384
NC = 2
NS = 16
NLANE = 16
CHUNK = 128
NCHUNK = NUPD // CHUNK
CPT = NCHUNK // NS
ROWS_PER_CORE = NROWS // NC
G = 8
NGROUP = ROWS_PER_CORE // G

COPY = 6256
COPY_LAST = NCOLS - (NS - 1) * COPY


def _body(x_hbm, idx_hbm, src_hbm, out_hbm,
          buf, idx_v, idx2_v, src_v, stage_v, sem, sem_b, sem_src):
  c = lax.axis_index("c")
  s = lax.axis_index("s")
  off = s * COPY
  last = s == NS - 1

  pltpu.sync_copy(idx_hbm.at[pl.ds(s * CPT, CPT)], idx_v)
  for i in range(G):
    for j in range(CPT):
      for k in range(CHUNK // NLANE):
        idx2_v[i * CPT + j, pl.ds(k * NLANE, NLANE)] = (
            idx_v[j, pl.ds(k * NLANE, NLANE)] + i * NCOLS)

  def dense_copy(mk_src, mk_dst, n, hop_sem):
    for i in range(G):
      pltpu.async_copy(mk_src(i, n), mk_dst(i, n), hop_sem)
    for i in range(G):
      pltpu.make_async_copy(mk_src(i, n), mk_dst(i, n), hop_sem).wait()

  def hop(mk_src, mk_dst, hop_sem):
    @pl.when(~last)
    def _():
      dense_copy(mk_src, mk_dst, COPY, hop_sem)

    @pl.when(last)
    def _():
      dense_copy(mk_src, mk_dst, COPY_LAST, hop_sem)

  def group_step(g, carry):
    r0 = c * ROWS_PER_CORE + g * G
    base = r0 * NCOLS

    x_at = lambda i, n: x_hbm.at[pl.ds(base + i * NCOLS + off, n)]
    out_at = lambda i, n: out_hbm.at[pl.ds(base + i * NCOLS + off, n)]
    stage_at = lambda i, n: stage_v.at[pl.ds(i * COPY, n)]
    buf_at = lambda i, n: buf.at[pl.ds(i * NCOLS + off, n)]

    for i in range(G):
      pltpu.async_copy(src_hbm.at[r0 + i, pl.ds(s * CPT, CPT)],
                       src_v.at[pl.ds(i * CPT, CPT)], sem_src)
    hop(x_at, stage_at, sem)
    hop(stage_at, buf_at, sem_b)
    for i in range(G):
      pltpu.make_async_copy(src_hbm.at[r0 + i, pl.ds(s * CPT, CPT)],
                            src_v.at[pl.ds(i * CPT, CPT)], sem_src).wait()

    plsc.subcore_barrier()

    for i in range(G):
      for j in range(CPT):
        pltpu.sync_copy(src_v.at[i * CPT + j],
                        buf.at[idx2_v.at[i * CPT + j]], add=True)

    plsc.subcore_barrier()

    hop(buf_at, stage_at, sem_b)
    hop(stage_at, out_at, sem)

    plsc.subcore_barrier()
    return carry

  lax.fori_loop(0, NGROUP, group_step, 0)


@jax.jit
def kernel(x, indices, src):
  idx = indices.astype(jnp.int32).reshape(NCHUNK, CHUNK)
  src3 = src.reshape(NROWS, NCHUNK, CHUNK)
  x_flat = x.reshape(NROWS * NCOLS)

  mesh = plsc.VectorSubcoreMesh(
      core_axis_name="c", subcore_axis_name="s", num_cores=NC, num_subcores=NS)
  run = pl.kernel(
      _body,
      out_type=jax.ShapeDtypeStruct((NROWS * NCOLS,), jnp.float32),
      mesh=mesh,
      scratch_types=[
          pltpu.VMEM_SHARED((G * NCOLS,), jnp.float32),
          pltpu.VMEM((CPT, CHUNK), jnp.int32),
          pltpu.VMEM((G * CPT, CHUNK), jnp.int32),
          pltpu.VMEM((G * CPT, CHUNK), jnp.float32),
          pltpu.VMEM((G * COPY,), jnp.float32),
          pltpu.SemaphoreType.DMA,
          pltpu.SemaphoreType.DMA,
          pltpu.SemaphoreType.DMA,
      ],
  )
  out = run(x_flat, idx, src3)
  return out.reshape(NROWS, NCOLS)


# speedup vs baseline: 1.3915x; 1.0975x over previous
import functools

import jax
import jax.numpy as jnp
from jax import lax
from jax.experimental import pallas as pl
from jax.experimental.pallas import tpu as pltpu
from jax.experimental.pallas import tpu_sc as plsc

NROWS = 128
NCOLS = 100000
NUPD = 16384
NC = 2
NS = 16
NLANE = 16
CHUNK = 128
NCHUNK = NUPD // CHUNK
CPT = NCHUNK // NS
ROWS_PER_CORE = NROWS // NC
G = 8
NGROUP = ROWS_PER_CORE // G

COPY = 6256
COPY_LAST = NCOLS - (NS - 1) * COPY


def _body(x_hbm, idx_hbm, src_hbm, out_hbm,
          buf, idx_v, idx2_v, src_v, stage_v, sem, sem_b, sem_src):
  c = lax.axis_index("c")
  s = lax.axis_index("s")
  off = s * COPY
  last = s == NS - 1

  pltpu.sync_copy(idx_hbm.at[pl.ds(s * CPT, CPT)], idx_v)
  for i in range(G):
    for j in range(CPT):
      for k in range(CHUNK // NLANE):
        idx2_v[i * CPT + j, pl.ds(k * NLANE, NLANE)] = (
            idx_v[j, pl.ds(k * NLANE, NLANE)] + i * NCOLS)

  def dense_copy(mk_src, mk_dst, n, hop_sem):
    for i in range(G):
      pltpu.async_copy(mk_src(i, n), mk_dst(i, n), hop_sem)
    for i in range(G):
      pltpu.make_async_copy(mk_src(i, n), mk_dst(i, n), hop_sem).wait()

  def hop(mk_src, mk_dst, hop_sem):
    @pl.when(~last)
    def _():
      dense_copy(mk_src, mk_dst, COPY, hop_sem)

    @pl.when(last)
    def _():
      dense_copy(mk_src, mk_dst, COPY_LAST, hop_sem)

  def group_step(g, carry):
    r0 = c * ROWS_PER_CORE + g * G
    base = r0 * NCOLS

    x_at = lambda i, n: x_hbm.at[pl.ds(base + i * NCOLS + off, n)]
    out_at = lambda i, n: out_hbm.at[pl.ds(base + i * NCOLS + off, n)]
    stage_at = lambda i, n: stage_v.at[pl.ds(i * COPY, n)]
    buf_at = lambda i, n: buf.at[pl.ds(i * NCOLS + off, n)]

    for i in range(G):
      pltpu.async_copy(src_hbm.at[r0 + i, pl.ds(s * CPT, CPT)],
                       src_v.at[pl.ds(i * CPT, CPT)], sem_src)
    hop(x_at, stage_at, sem)
    hop(stage_at, buf_at, sem_b)
    for i in range(G):
      pltpu.make_async_copy(src_hbm.at[r0 + i, pl.ds(s * CPT, CPT)],
                            src_v.at[pl.ds(i * CPT, CPT)], sem_src).wait()

    plsc.subcore_barrier()

    for i in range(G):
      for j in range(CPT):
        pltpu.async_copy(src_v.at[i * CPT + j],
                         buf.at[idx2_v.at[i * CPT + j]], sem_b, add=True)
    for i in range(G):
      for j in range(CPT):
        pltpu.make_async_copy(src_v.at[i * CPT + j],
                              buf.at[idx2_v.at[i * CPT + j]], sem_b).wait()

    plsc.subcore_barrier()

    hop(buf_at, stage_at, sem_b)
    hop(stage_at, out_at, sem)

    plsc.subcore_barrier()
    return carry

  lax.fori_loop(0, NGROUP, group_step, 0)


@jax.jit
def kernel(x, indices, src):
  idx = indices.astype(jnp.int32).reshape(NCHUNK, CHUNK)
  src3 = src.reshape(NROWS, NCHUNK, CHUNK)
  x_flat = x.reshape(NROWS * NCOLS)

  mesh = plsc.VectorSubcoreMesh(
      core_axis_name="c", subcore_axis_name="s", num_cores=NC, num_subcores=NS)
  run = pl.kernel(
      _body,
      out_type=jax.ShapeDtypeStruct((NROWS * NCOLS,), jnp.float32),
      mesh=mesh,
      scratch_types=[
          pltpu.VMEM_SHARED((G * NCOLS,), jnp.float32),
          pltpu.VMEM((CPT, CHUNK), jnp.int32),
          pltpu.VMEM((G * CPT, CHUNK), jnp.int32),
          pltpu.VMEM((G * CPT, CHUNK), jnp.float32),
          pltpu.VMEM((G * COPY,), jnp.float32),
          pltpu.SemaphoreType.DMA,
          pltpu.SemaphoreType.DMA,
          pltpu.SemaphoreType.DMA,
      ],
  )
  out = run(x_flat, idx, src3)
  return out.reshape(NROWS, NCOLS)


# speedup vs baseline: 1.5199x; 1.0923x over previous
import functools

import jax
import jax.numpy as jnp
from jax import lax
from jax.experimental import pallas as pl
from jax.experimental.pallas import tpu as pltpu
from jax.experimental.pallas import tpu_sc as plsc

NROWS = 128
NCOLS = 100000
NUPD = 16384
NC = 2
NS = 16
NLANE = 16
CHUNK = 128
NCHUNK = NUPD // CHUNK
CPT = NCHUNK // NS
ROWS_PER_CORE = NROWS // NC
G = 4
NGROUP = ROWS_PER_CORE // G

COPY = 6256
COPY_LAST = NCOLS - (NS - 1) * COPY


def _body(x_hbm, idx_hbm, src_hbm, out_hbm,
          buf0, buf1, idx_v, idx2_v, src_v0, src_v1, stage_v0, stage_v1,
          sem_ld, sem_xb, sem_sc, sem_out, sem_src):
  c = lax.axis_index("c")
  s = lax.axis_index("s")
  off = s * COPY
  last = s == NS - 1
  bufs = (buf0, buf1)
  stages = (stage_v0, stage_v1)
  srcs = (src_v0, src_v1)

  pltpu.sync_copy(idx_hbm.at[pl.ds(s * CPT, CPT)], idx_v)
  for i in range(G):
    for j in range(CPT):
      for k in range(CHUNK // NLANE):
        idx2_v[i * CPT + j, pl.ds(k * NLANE, NLANE)] = (
            idx_v[j, pl.ds(k * NLANE, NLANE)] + i * NCOLS)

  def row_base(g, i):
    return (c * ROWS_PER_CORE + g * G + i) * NCOLS + off

  def fire_x_loads(g, p, n):
    for i in range(G):
      pltpu.async_copy(x_hbm.at[pl.ds(row_base(g, i), n)],
                       stages[p].at[pl.ds(i * COPY, n)], sem_ld)

  def drain_x_loads(g, p, n):
    for i in range(G):
      pltpu.make_async_copy(x_hbm.at[pl.ds(row_base(g, i), n)],
                            stages[p].at[pl.ds(i * COPY, n)], sem_ld).wait()

  def stage_to_buf(p, n):
    for i in range(G):
      pltpu.async_copy(stages[p].at[pl.ds(i * COPY, n)],
                       bufs[p].at[pl.ds(i * NCOLS + off, n)], sem_xb)
    for i in range(G):
      pltpu.make_async_copy(stages[p].at[pl.ds(i * COPY, n)],
                            bufs[p].at[pl.ds(i * NCOLS + off, n)],
                            sem_xb).wait()

  def buf_to_stage(p, n):
    for i in range(G):
      pltpu.async_copy(bufs[p].at[pl.ds(i * NCOLS + off, n)],
                       stages[p].at[pl.ds(i * COPY, n)], sem_xb)
    for i in range(G):
      pltpu.make_async_copy(bufs[p].at[pl.ds(i * NCOLS + off, n)],
                            stages[p].at[pl.ds(i * COPY, n)], sem_xb).wait()

  def fire_out_writes(g, p, n):
    for i in range(G):
      pltpu.async_copy(stages[p].at[pl.ds(i * COPY, n)],
                       out_hbm.at[pl.ds(row_base(g, i), n)], sem_out)

  def drain_out_writes(g, p, n):
    for i in range(G):
      pltpu.make_async_copy(stages[p].at[pl.ds(i * COPY, n)],
                            out_hbm.at[pl.ds(row_base(g, i), n)],
                            sem_out).wait()

  def sized(fn, *args):
    @pl.when(~last)
    def _():
      fn(*args, COPY)

    @pl.when(last)
    def _():
      fn(*args, COPY_LAST)

  def fire_src_load(g, p):
    for i in range(G):
      pltpu.async_copy(
          src_hbm.at[c * ROWS_PER_CORE + g * G + i, pl.ds(s * CPT, CPT)],
          srcs[p].at[pl.ds(i * CPT, CPT)], sem_src)

  def drain_src_load(g, p):
    for i in range(G):
      pltpu.make_async_copy(
          src_hbm.at[c * ROWS_PER_CORE + g * G + i, pl.ds(s * CPT, CPT)],
          srcs[p].at[pl.ds(i * CPT, CPT)], sem_src).wait()

  fire_src_load(0, 0)
  sized(fire_x_loads, 0, 0)

  def group_step(g, carry):
    p = lax.rem(g, 2)

    def even_odd(fn):
      @pl.when(p == 0)
      def _():
        fn(0)

      @pl.when(p == 1)
      def _():
        fn(1)

    def stage_in(p):
      sized(drain_x_loads, g, p)
      sized(stage_to_buf, p)
      drain_src_load(g, p)

    even_odd(stage_in)

    plsc.subcore_barrier()

    def fire_scatter(p):
      for i in range(G):
        for j in range(CPT):
          pltpu.async_copy(srcs[p].at[i * CPT + j],
                           bufs[p].at[idx2_v.at[i * CPT + j]], sem_sc,
                           add=True)

    def drain_scatter(p):
      for i in range(G):
        for j in range(CPT):
          pltpu.make_async_copy(srcs[p].at[i * CPT + j],
                                bufs[p].at[idx2_v.at[i * CPT + j]],
                                sem_sc).wait()

    even_odd(fire_scatter)

    @pl.when(g + 1 < NGROUP)
    def _():
      def prefetch(pn):
        @pl.when(g >= 1)
        def _():
          sized(drain_out_writes, g - 1, pn)

        fire_src_load(g + 1, pn)
        sized(fire_x_loads, g + 1, pn)

      @pl.when(p == 0)
      def _():
        prefetch(1)

      @pl.when(p == 1)
      def _():
        prefetch(0)

    even_odd(drain_scatter)

    plsc.subcore_barrier()

    def write_back(p):
      sized(buf_to_stage, p)
      sized(fire_out_writes, g, p)

    even_odd(write_back)

    plsc.subcore_barrier()
    return carry

  lax.fori_loop(0, NGROUP, group_step, 0)

  p_last = (NGROUP - 1) % 2
  sized(drain_out_writes, NGROUP - 1, p_last)


@jax.jit
def kernel(x, indices, src):
  idx = indices.astype(jnp.int32).reshape(NCHUNK, CHUNK)
  src3 = src.reshape(NROWS, NCHUNK, CHUNK)
  x_flat = x.reshape(NROWS * NCOLS)

  mesh = plsc.VectorSubcoreMesh(
      core_axis_name="c", subcore_axis_name="s", num_cores=NC, num_subcores=NS)
  run = pl.kernel(
      _body,
      out_type=jax.ShapeDtypeStruct((NROWS * NCOLS,), jnp.float32),
      mesh=mesh,
      scratch_types=[
          pltpu.VMEM_SHARED((G * NCOLS,), jnp.float32),
          pltpu.VMEM_SHARED((G * NCOLS,), jnp.float32),
          pltpu.VMEM((CPT, CHUNK), jnp.int32),
          pltpu.VMEM((G * CPT, CHUNK), jnp.int32),
          pltpu.VMEM((G * CPT, CHUNK), jnp.float32),
          pltpu.VMEM((G * CPT, CHUNK), jnp.float32),
          pltpu.VMEM((G * COPY,), jnp.float32),
          pltpu.VMEM((G * COPY,), jnp.float32),
          pltpu.SemaphoreType.DMA,
          pltpu.SemaphoreType.DMA,
          pltpu.SemaphoreType.DMA,
          pltpu.SemaphoreType.DMA,
          pltpu.SemaphoreType.DMA,
      ],
  )
  out = run(x_flat, idx, src3)
  return out.reshape(NROWS, NCOLS)
